# baseline (speedup 1.0000x reference)
import functools

import jax
import jax.numpy as jnp
from jax import lax
from jax.experimental import pallas as pl
from jax.experimental.pallas import tpu as pltpu
from jax.experimental.pallas import tpu_sc as plsc

N = 5000
BLK = 512
NBLK = 10
NP = BLK * NBLK
NC = 512
K = 100
GK = 128
SIGMA = 0.5
SCORE_THRESH = 0.05


def _r2(f, x):
    return f(f(x, axis=0, keepdims=True), axis=1, keepdims=True)


def _rank_kernel(sc, sr, rank_ref):
    ones = jnp.ones((1, BLK), jnp.float32)
    tri = (lax.broadcasted_iota(jnp.int32, (BLK, 1), 0)
           < lax.broadcasted_iota(jnp.int32, (1, NC), 1))
    for jb in range(NBLK):
        srj = sr[jb:jb + 1, :]
        rnk = jnp.zeros((1, NC), jnp.float32)
        for ib in range(NBLK):
            scb = sc[ib * BLK:(ib + 1) * BLK, :]
            if ib < jb:
                mf = (scb >= srj).astype(jnp.float32)
            elif ib > jb:
                mf = (scb > srj).astype(jnp.float32)
            else:
                m = (scb > srj) | ((scb == srj) & tri)
                mf = m.astype(jnp.float32)
            rnk = rnk + jnp.dot(ones, mf,
                                preferred_element_type=jnp.float32)
        rank_ref[jb:jb + 1, :] = rnk.astype(jnp.int32)


def _rank_call(s_col, s_row):
    return pl.pallas_call(
        _rank_kernel,
        out_shape=jax.ShapeDtypeStruct((NBLK, NC), jnp.int32),
    )(s_col, s_row)


def _make_permute():
    mesh = plsc.VectorSubcoreMesh(core_axis_name="c", subcore_axis_name="s")

    @functools.partial(
        pl.kernel, mesh=mesh,
        out_type=jax.ShapeDtypeStruct((5 * NP,), jnp.float32),
        compiler_params=pltpu.CompilerParams(needs_layout_passes=False),
        scratch_types=[
            pltpu.VMEM((NP,), jnp.int32),
            pltpu.VMEM((NP,), jnp.float32),
            pltpu.VMEM((NP,), jnp.float32),
        ],
    )
    def permute_k(rank_hbm, vals_hbm, out_hbm, rank_v, seg_v, out_v):
        cid = lax.axis_index("c")
        sid = lax.axis_index("s")

        @pl.when((cid == 0) & (sid < 5))
        def _():
            base = sid * NP
            pltpu.sync_copy(rank_hbm, rank_v)
            pltpu.sync_copy(vals_hbm.at[pl.ds(base, NP)], seg_v)
            for g in range(NP // 16):
                idx = rank_v[pl.ds(g * 16, 16)]
                v = seg_v[pl.ds(g * 16, 16)]
                plsc.store_scatter(out_v, [idx], v)
            pltpu.sync_copy(out_v, out_hbm.at[pl.ds(base, NP)])

    return permute_k


def _tri_kernel(rowm, det_ref, dmax_ref):
    tri = (lax.broadcasted_iota(jnp.int32, (BLK, 1), 0)
           < lax.broadcasted_iota(jnp.int32, (1, NC), 1)).astype(jnp.float32)

    def rrow(a, jb):
        return rowm[a * NBLK + jb:a * NBLK + jb + 1, :]

    def ccol(a, ib):
        return jnp.transpose(rowm[a * NBLK + ib:a * NBLK + ib + 1, :])

    for jb in range(NBLK):
        x1r = rrow(0, jb)
        y1r = rrow(1, jb)
        x2r = rrow(2, jb)
        y2r = rrow(3, jb)
        arj = (x2r - x1r) * (y2r - y1r)

        acc = jnp.zeros((1, NC), jnp.float32)
        for ib in range(jb + 1):
            x1c = ccol(0, ib)
            y1c = ccol(1, ib)
            x2c = ccol(2, ib)
            y2c = ccol(3, ib)
            ac = (x2c - x1c) * (y2c - y1c)
            xx1 = jnp.maximum(x1c, x1r)
            yy1 = jnp.maximum(y1c, y1r)
            xx2 = jnp.minimum(x2c, x2r)
            yy2 = jnp.minimum(y2c, y2r)
            iw = jnp.maximum(xx2 - xx1, 0.0)
            inter = iw * (yy2 - yy1)
            union = ac + arj - inter
            iou = inter / (union + 1e-8)
            if ib == jb:
                iou = iou * tri
            acc = jnp.maximum(acc, jnp.max(iou, axis=0, keepdims=True))
        dmax_ref[jb:jb + 1, :] = acc

    m_all = dmax_ref[...]
    s_all = rowm[4 * NBLK:5 * NBLK, :]
    valid = s_all > -0.5
    draw = s_all * jnp.exp(-(m_all * m_all) / SIGMA)
    dthr = jnp.where(draw > SCORE_THRESH, draw, 0.0)
    d0 = jnp.where(valid, dthr, -1.0)
    code = (lax.broadcasted_iota(jnp.int32, (NBLK, NC), 0) * NC
            + lax.broadcasted_iota(jnp.int32, (NBLK, NC), 1))

    def pick(d, out, kk):
        mv = _r2(jnp.max, d)
        t1 = d == mv
        im = _r2(jnp.min, jnp.where(t1, code, jnp.int32(2 ** 30)))
        oh = t1 & (code == im)
        idxsel = im.astype(jnp.float32)
        rowi = lax.broadcasted_iota(jnp.int32, (8, 128), 0)
        lane = lax.broadcasted_iota(jnp.int32, (8, 128), 1)
        colv = jnp.where(rowi == 4, mv,
                         jnp.where(rowi == 5, idxsel, 0.0))
        out = out + jnp.where(lane == kk, colv, 0.0)
        d = jnp.where(oh, -2.0, d)
        return d, out

    def body(k, carry):
        d, out = carry
        for t in range(5):
            d, out = pick(d, out, 5 * k + t)
        return d, out

    _, out = lax.fori_loop(
        0, K // 5, body, (d0, jnp.zeros((8, 128), jnp.float32)))
    det_ref[...] = out


def _tri_call(rowm):
    return pl.pallas_call(
        _tri_kernel,
        out_shape=jax.ShapeDtypeStruct((8, 128), jnp.float32),
        scratch_shapes=[pltpu.VMEM((NBLK, NC), jnp.float32)],
        compiler_params=pltpu.CompilerParams(
            vmem_limit_bytes=100 * 1024 * 1024),
    )(rowm)


def _make_gather():
    mesh = plsc.VectorSubcoreMesh(core_axis_name="c", subcore_axis_name="s")

    @functools.partial(
        pl.kernel, mesh=mesh,
        out_type=jax.ShapeDtypeStruct((4 * GK,), jnp.float32),
        compiler_params=pltpu.CompilerParams(needs_layout_passes=False),
        scratch_types=[
            pltpu.VMEM((GK,), jnp.int32),
            pltpu.VMEM((4 * NP,), jnp.float32),
            pltpu.VMEM((4 * GK,), jnp.float32),
        ],
    )
    def gather_k(idx_hbm, flat_hbm, out_hbm, idx_v, flat_v, out_v):
        cid = lax.axis_index("c")
        sid = lax.axis_index("s")

        @pl.when((cid == 0) & (sid == 0))
        def _():
            pltpu.sync_copy(idx_hbm, idx_v)
            pltpu.sync_copy(flat_hbm.at[pl.ds(0, 4 * NP)], flat_v)
            for i in range(GK // 16):
                iv = idx_v[pl.ds(i * 16, 16)]
                for c in range(4):
                    vals = plsc.load_gather(flat_v, [iv + c * NP])
                    out_v[pl.ds(c * GK + i * 16, 16)] = vals
            pltpu.sync_copy(out_v, out_hbm)

    return gather_k


_permute_fn = None
_gather_fn = None


def _permute_vals(rank, vals):
    global _permute_fn
    if _permute_fn is None:
        _permute_fn = _make_permute()
    return _permute_fn(rank, vals)


def _gather_boxes(idx, flat):
    global _gather_fn
    if _gather_fn is None:
        _gather_fn = _make_gather()
    return _gather_fn(idx, flat)


def kernel(boxes, scores):
    boxes = boxes.astype(jnp.float32)
    scores = scores.astype(jnp.float32)
    padn = NP - N

    zp = jnp.zeros((padn,), jnp.float32)
    vals = jnp.concatenate([
        boxes[:, 0], zp,
        boxes[:, 1], zp,
        boxes[:, 2], zp,
        boxes[:, 3], zp,
        scores, jnp.full((padn,), -1.0, jnp.float32),
    ])
    s_np = vals[4 * NP:]

    rank_i = _rank_call(s_np[:, None], s_np.reshape(NBLK, NC)).reshape(-1)

    svals = _permute_vals(rank_i, vals)

    out = _tri_call(svals.reshape(5 * NBLK, NC))

    top_s = out[4, :K]
    idx = out[5, :].astype(jnp.int32)

    rows = _gather_boxes(idx, svals).reshape(4, GK).T
    return jnp.concatenate([rows[:K], top_s[:, None]], axis=1)


# speedup vs baseline: 1.0245x; 1.0245x over previous
import functools

import jax
import jax.numpy as jnp
from jax import lax
from jax.experimental import pallas as pl
from jax.experimental.pallas import tpu as pltpu
from jax.experimental.pallas import tpu_sc as plsc

N = 5000
BLK = 512
NBLK = 10
NP = BLK * NBLK
NC = 512
K = 100
GK = 128
SIGMA = 0.5
SCORE_THRESH = 0.05


def _r2(f, x):
    return f(f(x, axis=0, keepdims=True), axis=1, keepdims=True)


def _rank_kernel(sc, sr, rank_ref):
    ones = jnp.ones((1, BLK), jnp.float32)
    tri = (lax.broadcasted_iota(jnp.int32, (BLK, 1), 0)
           < lax.broadcasted_iota(jnp.int32, (1, NC), 1))
    for jb in range(NBLK):
        srj = sr[jb:jb + 1, :]
        rnk = jnp.zeros((1, NC), jnp.float32)
        for ib in range(NBLK):
            scb = sc[ib * BLK:(ib + 1) * BLK, :]
            if ib < jb:
                mf = (scb >= srj).astype(jnp.float32)
            elif ib > jb:
                mf = (scb > srj).astype(jnp.float32)
            else:
                m = (scb > srj) | ((scb == srj) & tri)
                mf = m.astype(jnp.float32)
            rnk = rnk + jnp.dot(ones, mf,
                                preferred_element_type=jnp.float32)
        rank_ref[jb:jb + 1, :] = rnk.astype(jnp.int32)


def _rank_call(s_col, s_row):
    return pl.pallas_call(
        _rank_kernel,
        out_shape=jax.ShapeDtypeStruct((NBLK, NC), jnp.int32),
    )(s_col, s_row)


def _make_permute():
    mesh = plsc.VectorSubcoreMesh(core_axis_name="c", subcore_axis_name="s")

    @functools.partial(
        pl.kernel, mesh=mesh,
        out_type=jax.ShapeDtypeStruct((5 * NP,), jnp.float32),
        compiler_params=pltpu.CompilerParams(needs_layout_passes=False),
        scratch_types=[
            pltpu.VMEM((NP,), jnp.int32),
            pltpu.VMEM((NP,), jnp.float32),
            pltpu.VMEM((NP,), jnp.float32),
        ],
    )
    def permute_k(rank_hbm, vals_hbm, out_hbm, rank_v, seg_v, out_v):
        cid = lax.axis_index("c")
        sid = lax.axis_index("s")

        @pl.when((cid == 0) & (sid < 5))
        def _():
            base = sid * NP
            pltpu.sync_copy(rank_hbm, rank_v)
            pltpu.sync_copy(vals_hbm.at[pl.ds(base, NP)], seg_v)
            for g in range(NP // 16):
                idx = rank_v[pl.ds(g * 16, 16)]
                v = seg_v[pl.ds(g * 16, 16)]
                plsc.store_scatter(out_v, [idx], v)
            pltpu.sync_copy(out_v, out_hbm.at[pl.ds(base, NP)])

    return permute_k


def _tri_kernel(col, rowm, det_ref, dmax_ref):
    tri = (lax.broadcasted_iota(jnp.int32, (BLK, 1), 0)
           < lax.broadcasted_iota(jnp.int32, (1, NC), 1)).astype(jnp.float32)

    def rrow(a, jb):
        return rowm[a * NBLK + jb:a * NBLK + jb + 1, :]

    def ccol(a, ib):
        return col[a * NP + ib * BLK:a * NP + (ib + 1) * BLK, :]

    for jb in range(NBLK):
        x1r = rrow(0, jb)
        y1r = rrow(1, jb)
        x2r = rrow(2, jb)
        y2r = rrow(3, jb)
        arj = (x2r - x1r) * (y2r - y1r)

        acc = jnp.zeros((1, NC), jnp.float32)
        for ib in range(jb + 1):
            x1c = ccol(0, ib)
            y1c = ccol(1, ib)
            x2c = ccol(2, ib)
            y2c = ccol(3, ib)
            ac = (x2c - x1c) * (y2c - y1c)
            xx1 = jnp.maximum(x1c, x1r)
            yy1 = jnp.maximum(y1c, y1r)
            xx2 = jnp.minimum(x2c, x2r)
            yy2 = jnp.minimum(y2c, y2r)
            iw = jnp.maximum(xx2 - xx1, 0.0)
            inter = iw * (yy2 - yy1)
            union = ac + arj - inter
            iou = inter / (union + 1e-8)
            if ib == jb:
                iou = iou * tri
            acc = jnp.maximum(acc, jnp.max(iou, axis=0, keepdims=True))
        dmax_ref[jb:jb + 1, :] = acc

    m_all = dmax_ref[...]
    s_all = rowm[4 * NBLK:5 * NBLK, :]
    valid = s_all > -0.5
    draw = s_all * jnp.exp(-(m_all * m_all) / SIGMA)
    dthr = jnp.where(draw > SCORE_THRESH, draw, 0.0)
    d0 = jnp.where(valid, dthr, -1.0)
    code = (lax.broadcasted_iota(jnp.int32, (NBLK, NC), 0) * NC
            + lax.broadcasted_iota(jnp.int32, (NBLK, NC), 1))

    def pick(d, out, kk):
        mv = _r2(jnp.max, d)
        t1 = d == mv
        im = _r2(jnp.min, jnp.where(t1, code, jnp.int32(2 ** 30)))
        oh = t1 & (code == im)
        idxsel = im.astype(jnp.float32)
        rowi = lax.broadcasted_iota(jnp.int32, (8, 128), 0)
        lane = lax.broadcasted_iota(jnp.int32, (8, 128), 1)
        colv = jnp.where(rowi == 4, mv,
                         jnp.where(rowi == 5, idxsel, 0.0))
        out = out + jnp.where(lane == kk, colv, 0.0)
        d = jnp.where(oh, -2.0, d)
        return d, out

    def body(k, carry):
        d, out = carry
        for t in range(5):
            d, out = pick(d, out, 5 * k + t)
        return d, out

    _, out = lax.fori_loop(
        0, K // 5, body, (d0, jnp.zeros((8, 128), jnp.float32)))
    det_ref[...] = out


def _tri_call(col, rowm):
    return pl.pallas_call(
        _tri_kernel,
        out_shape=jax.ShapeDtypeStruct((8, 128), jnp.float32),
        scratch_shapes=[pltpu.VMEM((NBLK, NC), jnp.float32)],
    )(col, rowm)


def _make_gather():
    mesh = plsc.VectorSubcoreMesh(core_axis_name="c", subcore_axis_name="s")

    @functools.partial(
        pl.kernel, mesh=mesh,
        out_type=jax.ShapeDtypeStruct((4 * GK,), jnp.float32),
        compiler_params=pltpu.CompilerParams(needs_layout_passes=False),
        scratch_types=[
            pltpu.VMEM((GK,), jnp.int32),
            pltpu.VMEM((4 * NP,), jnp.float32),
            pltpu.VMEM((4 * GK,), jnp.float32),
        ],
    )
    def gather_k(idx_hbm, flat_hbm, out_hbm, idx_v, flat_v, out_v):
        cid = lax.axis_index("c")
        sid = lax.axis_index("s")

        @pl.when((cid == 0) & (sid == 0))
        def _():
            pltpu.sync_copy(idx_hbm, idx_v)
            pltpu.sync_copy(flat_hbm.at[pl.ds(0, 4 * NP)], flat_v)
            for i in range(GK // 16):
                iv = idx_v[pl.ds(i * 16, 16)]
                for c in range(4):
                    vals = plsc.load_gather(flat_v, [iv + c * NP])
                    out_v[pl.ds(c * GK + i * 16, 16)] = vals
            pltpu.sync_copy(out_v, out_hbm)

    return gather_k


_permute_fn = None
_gather_fn = None


def _permute_vals(rank, vals):
    global _permute_fn
    if _permute_fn is None:
        _permute_fn = _make_permute()
    return _permute_fn(rank, vals)


def _gather_boxes(idx, flat):
    global _gather_fn
    if _gather_fn is None:
        _gather_fn = _make_gather()
    return _gather_fn(idx, flat)


def kernel(boxes, scores):
    boxes = boxes.astype(jnp.float32)
    scores = scores.astype(jnp.float32)
    padn = NP - N

    zp = jnp.zeros((padn,), jnp.float32)
    vals = jnp.concatenate([
        boxes[:, 0], zp,
        boxes[:, 1], zp,
        boxes[:, 2], zp,
        boxes[:, 3], zp,
        scores, jnp.full((padn,), -1.0, jnp.float32),
    ])
    s_np = vals[4 * NP:]

    rank_i = _rank_call(s_np[:, None], s_np.reshape(NBLK, NC)).reshape(-1)

    svals = _permute_vals(rank_i, vals)

    out = _tri_call(svals[:, None], svals.reshape(5 * NBLK, NC))

    top_s = out[4, :K]
    idx = out[5, :].astype(jnp.int32)

    rows = _gather_boxes(idx, svals).reshape(4, GK).T
    return jnp.concatenate([rows[:K], top_s[:, None]], axis=1)


# speedup vs baseline: 1.0450x; 1.0201x over previous
import functools

import jax
import jax.numpy as jnp
from jax import lax
from jax.experimental import pallas as pl
from jax.experimental.pallas import tpu as pltpu
from jax.experimental.pallas import tpu_sc as plsc

N = 5000
BLK = 512
NBLK = 10
NP = BLK * NBLK
NC = 512
K = 100
GK = 128
SIGMA = 0.5
SCORE_THRESH = 0.05


def _r2(f, x):
    return f(f(x, axis=0, keepdims=True), axis=1, keepdims=True)


def _rank_kernel(sr, rank_ref):
    ones = jnp.ones((1, BLK), jnp.float32)
    tri = (lax.broadcasted_iota(jnp.int32, (BLK, 1), 0)
           < lax.broadcasted_iota(jnp.int32, (1, NC), 1))
    scol = [jnp.transpose(sr[ib:ib + 1, :]) for ib in range(NBLK)]
    for jb in range(NBLK):
        srj = sr[jb:jb + 1, :]
        rnk = jnp.zeros((1, NC), jnp.float32)
        for ib in range(NBLK):
            scb = scol[ib]
            if ib < jb:
                mf = (scb >= srj).astype(jnp.float32)
            elif ib > jb:
                mf = (scb > srj).astype(jnp.float32)
            else:
                m = (scb > srj) | ((scb == srj) & tri)
                mf = m.astype(jnp.float32)
            rnk = rnk + jnp.dot(ones, mf,
                                preferred_element_type=jnp.float32)
        rank_ref[jb:jb + 1, :] = rnk.astype(jnp.int32)


def _rank_call(s_row):
    return pl.pallas_call(
        _rank_kernel,
        out_shape=jax.ShapeDtypeStruct((NBLK, NC), jnp.int32),
    )(s_row)


def _make_permute():
    mesh = plsc.VectorSubcoreMesh(core_axis_name="c", subcore_axis_name="s")

    @functools.partial(
        pl.kernel, mesh=mesh,
        out_type=jax.ShapeDtypeStruct((5 * NP,), jnp.float32),
        compiler_params=pltpu.CompilerParams(needs_layout_passes=False),
        scratch_types=[
            pltpu.VMEM((NP,), jnp.int32),
            pltpu.VMEM((NP,), jnp.float32),
            pltpu.VMEM((NP,), jnp.float32),
        ],
    )
    def permute_k(rank_hbm, vals_hbm, out_hbm, rank_v, seg_v, out_v):
        cid = lax.axis_index("c")
        sid = lax.axis_index("s")

        @pl.when((cid == 0) & (sid < 5))
        def _():
            base = sid * NP
            pltpu.sync_copy(rank_hbm, rank_v)
            pltpu.sync_copy(vals_hbm.at[pl.ds(base, NP)], seg_v)
            for g in range(NP // 16):
                idx = rank_v[pl.ds(g * 16, 16)]
                v = seg_v[pl.ds(g * 16, 16)]
                plsc.store_scatter(out_v, [idx], v)
            pltpu.sync_copy(out_v, out_hbm.at[pl.ds(base, NP)])

    return permute_k


def _tri_kernel(col, rowm, det_ref, dmax_ref):
    tri = (lax.broadcasted_iota(jnp.int32, (BLK, 1), 0)
           < lax.broadcasted_iota(jnp.int32, (1, NC), 1)).astype(jnp.float32)

    def rrow(a, jb):
        return rowm[a * NBLK + jb:a * NBLK + jb + 1, :]

    def ccol(a, ib):
        return col[a * NP + ib * BLK:a * NP + (ib + 1) * BLK, :]

    for jb in range(NBLK):
        x1r = rrow(0, jb)
        y1r = rrow(1, jb)
        x2r = rrow(2, jb)
        y2r = rrow(3, jb)
        arj = (x2r - x1r) * (y2r - y1r)

        acc = jnp.zeros((1, NC), jnp.float32)
        for ib in range(jb + 1):
            x1c = ccol(0, ib)
            y1c = ccol(1, ib)
            x2c = ccol(2, ib)
            y2c = ccol(3, ib)
            ac = (x2c - x1c) * (y2c - y1c)
            xx1 = jnp.maximum(x1c, x1r)
            yy1 = jnp.maximum(y1c, y1r)
            xx2 = jnp.minimum(x2c, x2r)
            yy2 = jnp.minimum(y2c, y2r)
            iw = jnp.maximum(xx2 - xx1, 0.0)
            inter = iw * (yy2 - yy1)
            union = ac + arj - inter
            iou = inter / (union + 1e-8)
            if ib == jb:
                iou = iou * tri
            acc = jnp.maximum(acc, jnp.max(iou, axis=0, keepdims=True))
        dmax_ref[jb:jb + 1, :] = acc

    m_all = dmax_ref[...]
    s_all = rowm[4 * NBLK:5 * NBLK, :]
    valid = s_all > -0.5
    draw = s_all * jnp.exp(-(m_all * m_all) / SIGMA)
    dthr = jnp.where(draw > SCORE_THRESH, draw, 0.0)
    d0 = jnp.where(valid, dthr, -1.0)
    code = (lax.broadcasted_iota(jnp.int32, (NBLK, NC), 0) * NC
            + lax.broadcasted_iota(jnp.int32, (NBLK, NC), 1))

    def pick(d, out, kk):
        mv = _r2(jnp.max, d)
        t1 = d == mv
        im = _r2(jnp.min, jnp.where(t1, code, jnp.int32(2 ** 30)))
        oh = t1 & (code == im)
        idxsel = im.astype(jnp.float32)
        rowi = lax.broadcasted_iota(jnp.int32, (8, 128), 0)
        lane = lax.broadcasted_iota(jnp.int32, (8, 128), 1)
        colv = jnp.where(rowi == 4, mv,
                         jnp.where(rowi == 5, idxsel, 0.0))
        out = out + jnp.where(lane == kk, colv, 0.0)
        d = jnp.where(oh, -2.0, d)
        return d, out

    def body(k, carry):
        d, out = carry
        for t in range(5):
            d, out = pick(d, out, 5 * k + t)
        return d, out

    _, out = lax.fori_loop(
        0, K // 5, body, (d0, jnp.zeros((8, 128), jnp.float32)))
    det_ref[...] = out


def _tri_call(col, rowm):
    return pl.pallas_call(
        _tri_kernel,
        out_shape=jax.ShapeDtypeStruct((8, 128), jnp.float32),
        scratch_shapes=[pltpu.VMEM((NBLK, NC), jnp.float32)],
    )(col, rowm)


def _make_gather():
    mesh = plsc.VectorSubcoreMesh(core_axis_name="c", subcore_axis_name="s")

    @functools.partial(
        pl.kernel, mesh=mesh,
        out_type=jax.ShapeDtypeStruct((4 * GK,), jnp.float32),
        compiler_params=pltpu.CompilerParams(needs_layout_passes=False),
        scratch_types=[
            pltpu.VMEM((GK,), jnp.int32),
            pltpu.VMEM((4 * NP,), jnp.float32),
            pltpu.VMEM((4 * GK,), jnp.float32),
        ],
    )
    def gather_k(idx_hbm, flat_hbm, out_hbm, idx_v, flat_v, out_v):
        cid = lax.axis_index("c")
        sid = lax.axis_index("s")

        @pl.when((cid == 0) & (sid == 0))
        def _():
            pltpu.sync_copy(idx_hbm, idx_v)
            pltpu.sync_copy(flat_hbm.at[pl.ds(0, 4 * NP)], flat_v)
            for i in range(GK // 16):
                iv = idx_v[pl.ds(i * 16, 16)]
                for c in range(4):
                    vals = plsc.load_gather(flat_v, [iv + c * NP])
                    out_v[pl.ds(c * GK + i * 16, 16)] = vals
            pltpu.sync_copy(out_v, out_hbm)

    return gather_k


_permute_fn = None
_gather_fn = None


def _permute_vals(rank, vals):
    global _permute_fn
    if _permute_fn is None:
        _permute_fn = _make_permute()
    return _permute_fn(rank, vals)


def _gather_boxes(idx, flat):
    global _gather_fn
    if _gather_fn is None:
        _gather_fn = _make_gather()
    return _gather_fn(idx, flat)


def kernel(boxes, scores):
    boxes = boxes.astype(jnp.float32)
    scores = scores.astype(jnp.float32)
    padn = NP - N

    zp = jnp.zeros((padn,), jnp.float32)
    vals = jnp.concatenate([
        boxes[:, 0], zp,
        boxes[:, 1], zp,
        boxes[:, 2], zp,
        boxes[:, 3], zp,
        scores, jnp.full((padn,), -1.0, jnp.float32),
    ])
    s_np = vals[4 * NP:]

    rank_i = _rank_call(s_np.reshape(NBLK, NC)).reshape(-1)

    svals = _permute_vals(rank_i, vals)

    out = _tri_call(svals[:, None], svals.reshape(5 * NBLK, NC))

    top_s = out[4, :K]
    idx = out[5, :].astype(jnp.int32)

    rows = _gather_boxes(idx, svals).reshape(4, GK).T
    return jnp.concatenate([rows[:K], top_s[:, None]], axis=1)


# speedup vs baseline: 1.1242x; 1.0758x over previous
import functools

import jax
import jax.numpy as jnp
from jax import lax
from jax.experimental import pallas as pl
from jax.experimental.pallas import tpu as pltpu
from jax.experimental.pallas import tpu_sc as plsc

N = 5000
BLK = 512
NBLK = 10
NP = BLK * NBLK
NC = 512
K = 100
GK = 128
SIGMA = 0.5
SCORE_THRESH = 0.05


def _r2(f, x):
    return f(f(x, axis=0, keepdims=True), axis=1, keepdims=True)


def _rank_kernel(sr, rank_ref):
    ones = jnp.ones((1, BLK), jnp.float32)
    tri = (lax.broadcasted_iota(jnp.int32, (BLK, 1), 0)
           < lax.broadcasted_iota(jnp.int32, (1, NC), 1))
    scol = [jnp.transpose(sr[ib:ib + 1, :]) for ib in range(NBLK)]
    for jb in range(NBLK):
        srj = sr[jb:jb + 1, :]
        rnk = jnp.zeros((1, NC), jnp.float32)
        for ib in range(NBLK):
            scb = scol[ib]
            if ib < jb:
                mf = (scb >= srj).astype(jnp.float32)
            elif ib > jb:
                mf = (scb > srj).astype(jnp.float32)
            else:
                m = (scb > srj) | ((scb == srj) & tri)
                mf = m.astype(jnp.float32)
            rnk = rnk + jnp.dot(ones, mf,
                                preferred_element_type=jnp.float32)
        rank_ref[jb:jb + 1, :] = rnk.astype(jnp.int32)


def _rank_call(s_row):
    return pl.pallas_call(
        _rank_kernel,
        out_shape=jax.ShapeDtypeStruct((NBLK, NC), jnp.int32),
    )(s_row)


def _make_permute():
    mesh = plsc.VectorSubcoreMesh(core_axis_name="c", subcore_axis_name="s")

    @functools.partial(
        pl.kernel, mesh=mesh,
        out_type=jax.ShapeDtypeStruct((5 * NP,), jnp.float32),
        compiler_params=pltpu.CompilerParams(needs_layout_passes=False),
        scratch_types=[
            pltpu.VMEM((NP,), jnp.int32),
            pltpu.VMEM((NP,), jnp.float32),
            pltpu.VMEM((NP,), jnp.float32),
        ],
    )
    def permute_k(rank_hbm, vals_hbm, out_hbm, rank_v, seg_v, out_v):
        cid = lax.axis_index("c")
        sid = lax.axis_index("s")

        @pl.when((cid == 0) & (sid < 5))
        def _():
            base = sid * NP
            pltpu.sync_copy(rank_hbm, rank_v)
            pltpu.sync_copy(vals_hbm.at[pl.ds(base, NP)], seg_v)
            for g in range(NP // 16):
                idx = rank_v[pl.ds(g * 16, 16)]
                v = seg_v[pl.ds(g * 16, 16)]
                plsc.store_scatter(out_v, [idx], v)
            pltpu.sync_copy(out_v, out_hbm.at[pl.ds(base, NP)])

    return permute_k


def _tri_kernel(rowm, det_ref, dmax_ref):
    tri = (lax.broadcasted_iota(jnp.int32, (BLK, 1), 0)
           < lax.broadcasted_iota(jnp.int32, (1, NC), 1)).astype(jnp.float32)

    def rrow(a, jb):
        return rowm[a * NBLK + jb:a * NBLK + jb + 1, :]

    tcol = [[jnp.transpose(rrow(a, ib)) for a in range(4)]
            for ib in range(NBLK)]
    acl = [(t[2] - t[0]) * (t[3] - t[1]) for t in tcol]

    for jb in range(NBLK):
        x1r = rrow(0, jb)
        y1r = rrow(1, jb)
        x2r = rrow(2, jb)
        y2r = rrow(3, jb)
        arj = (x2r - x1r) * (y2r - y1r)

        acc = jnp.zeros((1, NC), jnp.float32)
        for ib in range(jb + 1):
            x1c, y1c, x2c, y2c = tcol[ib]
            ac = acl[ib]
            xx1 = jnp.maximum(x1c, x1r)
            yy1 = jnp.maximum(y1c, y1r)
            xx2 = jnp.minimum(x2c, x2r)
            yy2 = jnp.minimum(y2c, y2r)
            iw = jnp.maximum(xx2 - xx1, 0.0)
            inter = iw * (yy2 - yy1)
            union = ac + arj - inter
            iou = inter / (union + 1e-8)
            if ib == jb:
                iou = iou * tri
            acc = jnp.maximum(acc, jnp.max(iou, axis=0, keepdims=True))
        dmax_ref[jb:jb + 1, :] = acc

    m_all = dmax_ref[...]
    s_all = rowm[4 * NBLK:5 * NBLK, :]
    valid = s_all > -0.5
    draw = s_all * jnp.exp(-(m_all * m_all) / SIGMA)
    dthr = jnp.where(draw > SCORE_THRESH, draw, 0.0)
    d0 = jnp.where(valid, dthr, -1.0)
    code = (lax.broadcasted_iota(jnp.int32, (NBLK, NC), 0) * NC
            + lax.broadcasted_iota(jnp.int32, (NBLK, NC), 1))

    def pick(d, out, kk):
        mv = _r2(jnp.max, d)
        t1 = d == mv
        im = _r2(jnp.min, jnp.where(t1, code, jnp.int32(2 ** 30)))
        oh = t1 & (code == im)
        idxsel = im.astype(jnp.float32)
        rowi = lax.broadcasted_iota(jnp.int32, (8, 128), 0)
        lane = lax.broadcasted_iota(jnp.int32, (8, 128), 1)
        colv = jnp.where(rowi == 4, mv,
                         jnp.where(rowi == 5, idxsel, 0.0))
        out = out + jnp.where(lane == kk, colv, 0.0)
        d = jnp.where(oh, -2.0, d)
        return d, out

    def body(k, carry):
        d, out = carry
        for t in range(5):
            d, out = pick(d, out, 5 * k + t)
        return d, out

    _, out = lax.fori_loop(
        0, K // 5, body, (d0, jnp.zeros((8, 128), jnp.float32)))
    det_ref[...] = out


def _tri_call(rowm):
    return pl.pallas_call(
        _tri_kernel,
        out_shape=jax.ShapeDtypeStruct((8, 128), jnp.float32),
        scratch_shapes=[pltpu.VMEM((NBLK, NC), jnp.float32)],
        compiler_params=pltpu.CompilerParams(
            vmem_limit_bytes=100 * 1024 * 1024),
    )(rowm)


def _make_gather():
    mesh = plsc.VectorSubcoreMesh(core_axis_name="c", subcore_axis_name="s")

    @functools.partial(
        pl.kernel, mesh=mesh,
        out_type=jax.ShapeDtypeStruct((4 * GK,), jnp.float32),
        compiler_params=pltpu.CompilerParams(needs_layout_passes=False),
        scratch_types=[
            pltpu.VMEM((GK,), jnp.int32),
            pltpu.VMEM((4 * NP,), jnp.float32),
            pltpu.VMEM((4 * GK,), jnp.float32),
        ],
    )
    def gather_k(idx_hbm, flat_hbm, out_hbm, idx_v, flat_v, out_v):
        cid = lax.axis_index("c")
        sid = lax.axis_index("s")

        @pl.when((cid == 0) & (sid == 0))
        def _():
            pltpu.sync_copy(idx_hbm, idx_v)
            pltpu.sync_copy(flat_hbm.at[pl.ds(0, 4 * NP)], flat_v)
            for i in range(GK // 16):
                iv = idx_v[pl.ds(i * 16, 16)]
                for c in range(4):
                    vals = plsc.load_gather(flat_v, [iv + c * NP])
                    out_v[pl.ds(c * GK + i * 16, 16)] = vals
            pltpu.sync_copy(out_v, out_hbm)

    return gather_k


_permute_fn = None
_gather_fn = None


def _permute_vals(rank, vals):
    global _permute_fn
    if _permute_fn is None:
        _permute_fn = _make_permute()
    return _permute_fn(rank, vals)


def _gather_boxes(idx, flat):
    global _gather_fn
    if _gather_fn is None:
        _gather_fn = _make_gather()
    return _gather_fn(idx, flat)


def kernel(boxes, scores):
    boxes = boxes.astype(jnp.float32)
    scores = scores.astype(jnp.float32)
    padn = NP - N

    zp = jnp.zeros((padn,), jnp.float32)
    vals = jnp.concatenate([
        boxes[:, 0], zp,
        boxes[:, 1], zp,
        boxes[:, 2], zp,
        boxes[:, 3], zp,
        scores, jnp.full((padn,), -1.0, jnp.float32),
    ])
    s_np = vals[4 * NP:]

    rank_i = _rank_call(s_np.reshape(NBLK, NC)).reshape(-1)

    svals = _permute_vals(rank_i, vals)

    out = _tri_call(svals.reshape(5 * NBLK, NC))

    top_s = out[4, :K]
    idx = out[5, :].astype(jnp.int32)

    rows = _gather_boxes(idx, svals).reshape(4, GK).T
    return jnp.concatenate([rows[:K], top_s[:, None]], axis=1)


# speedup vs baseline: 1.1442x; 1.0178x over previous
import functools

import jax
import jax.numpy as jnp
from jax import lax
from jax.experimental import pallas as pl
from jax.experimental.pallas import tpu as pltpu
from jax.experimental.pallas import tpu_sc as plsc

N = 5000
BLK = 512
NBLK = 10
NP = BLK * NBLK
NC = 512
K = 100
GK = 128
SIGMA = 0.5
SCORE_THRESH = 0.05


def _r2(f, x):
    return f(f(x, axis=0, keepdims=True), axis=1, keepdims=True)


def _rank_kernel(sr, rank_ref):
    ones = jnp.ones((1, BLK), jnp.float32)
    tri = (lax.broadcasted_iota(jnp.int32, (BLK, 1), 0)
           < lax.broadcasted_iota(jnp.int32, (1, NC), 1))
    scol = [jnp.transpose(sr[ib:ib + 1, :]) for ib in range(NBLK)]
    for jb in range(NBLK):
        srj = sr[jb:jb + 1, :]
        rnk = jnp.zeros((1, NC), jnp.float32)
        for ib in range(NBLK):
            scb = scol[ib]
            if ib < jb:
                mf = (scb >= srj).astype(jnp.float32)
            elif ib > jb:
                mf = (scb > srj).astype(jnp.float32)
            else:
                m = (scb > srj) | ((scb == srj) & tri)
                mf = m.astype(jnp.float32)
            rnk = rnk + jnp.dot(ones, mf,
                                preferred_element_type=jnp.float32)
        rank_ref[pl.ds(jb * BLK, BLK)] = jnp.reshape(
            rnk.astype(jnp.int32), (BLK,))


def _rank_call(s_row):
    return pl.pallas_call(
        _rank_kernel,
        out_shape=jax.ShapeDtypeStruct((NP,), jnp.int32),
    )(s_row)


def _make_permute():
    mesh = plsc.VectorSubcoreMesh(core_axis_name="c", subcore_axis_name="s")

    @functools.partial(
        pl.kernel, mesh=mesh,
        out_type=jax.ShapeDtypeStruct((5 * NP,), jnp.float32),
        compiler_params=pltpu.CompilerParams(needs_layout_passes=False),
        scratch_types=[
            pltpu.VMEM((NP,), jnp.int32),
            pltpu.VMEM((NP,), jnp.float32),
            pltpu.VMEM((NP,), jnp.float32),
            pltpu.SemaphoreType.DMA,
            pltpu.SemaphoreType.DMA,
        ],
    )
    def permute_k(rank_hbm, vals_hbm, out_hbm, rank_v, seg_v, out_v,
                  sem1, sem2):
        cid = lax.axis_index("c")
        sid = lax.axis_index("s")

        @pl.when((cid == 0) & (sid < 5))
        def _():
            base = sid * NP
            c1 = pltpu.async_copy(rank_hbm, rank_v, sem1)
            c2 = pltpu.async_copy(vals_hbm.at[pl.ds(base, NP)], seg_v, sem2)
            c1.wait()
            c2.wait()
            for g in range(NP // 16):
                idx = rank_v[pl.ds(g * 16, 16)]
                v = seg_v[pl.ds(g * 16, 16)]
                plsc.store_scatter(out_v, [idx], v)
            pltpu.sync_copy(out_v, out_hbm.at[pl.ds(base, NP)])

    return permute_k


def _tri_kernel(rowm, det_ref, dmax_ref):
    tri = (lax.broadcasted_iota(jnp.int32, (BLK, 1), 0)
           < lax.broadcasted_iota(jnp.int32, (1, NC), 1)).astype(jnp.float32)

    def rrow(a, jb):
        return rowm[a * NBLK + jb:a * NBLK + jb + 1, :]

    tcol = [[jnp.transpose(rrow(a, ib)) for a in range(4)]
            for ib in range(NBLK)]
    acl = [(t[2] - t[0]) * (t[3] - t[1]) for t in tcol]

    for jb in range(NBLK):
        x1r = rrow(0, jb)
        y1r = rrow(1, jb)
        x2r = rrow(2, jb)
        y2r = rrow(3, jb)
        arj = (x2r - x1r) * (y2r - y1r)

        acc = jnp.zeros((1, NC), jnp.float32)
        for ib in range(jb + 1):
            x1c, y1c, x2c, y2c = tcol[ib]
            ac = acl[ib]
            xx1 = jnp.maximum(x1c, x1r)
            yy1 = jnp.maximum(y1c, y1r)
            xx2 = jnp.minimum(x2c, x2r)
            yy2 = jnp.minimum(y2c, y2r)
            iw = jnp.maximum(xx2 - xx1, 0.0)
            inter = iw * (yy2 - yy1)
            union = ac + arj - inter
            iou = inter / (union + 1e-8)
            if ib == jb:
                iou = iou * tri
            acc = jnp.maximum(acc, jnp.max(iou, axis=0, keepdims=True))
        dmax_ref[jb:jb + 1, :] = acc

    m_all = dmax_ref[...]
    s_all = rowm[4 * NBLK:5 * NBLK, :]
    valid = s_all > -0.5
    draw = s_all * jnp.exp(-(m_all * m_all) / SIGMA)
    dthr = jnp.where(draw > SCORE_THRESH, draw, 0.0)
    d0 = jnp.where(valid, dthr, -1.0)
    code = (lax.broadcasted_iota(jnp.int32, (NBLK, NC), 0) * NC
            + lax.broadcasted_iota(jnp.int32, (NBLK, NC), 1))

    def pick(d, out, kk):
        mv = _r2(jnp.max, d)
        t1 = d == mv
        im = _r2(jnp.min, jnp.where(t1, code, jnp.int32(2 ** 30)))
        oh = t1 & (code == im)
        idxsel = im.astype(jnp.float32)
        rowi = lax.broadcasted_iota(jnp.int32, (8, 128), 0)
        lane = lax.broadcasted_iota(jnp.int32, (8, 128), 1)
        colv = jnp.where(rowi == 4, mv,
                         jnp.where(rowi == 5, idxsel, 0.0))
        out = out + jnp.where(lane == kk, colv, 0.0)
        d = jnp.where(oh, -2.0, d)
        return d, out

    def body(k, carry):
        d, out = carry
        for t in range(5):
            d, out = pick(d, out, 5 * k + t)
        return d, out

    _, out = lax.fori_loop(
        0, K // 5, body, (d0, jnp.zeros((8, 128), jnp.float32)))
    det_ref[...] = out


def _tri_call(rowm):
    return pl.pallas_call(
        _tri_kernel,
        out_shape=jax.ShapeDtypeStruct((8, 128), jnp.float32),
        scratch_shapes=[pltpu.VMEM((NBLK, NC), jnp.float32)],
        compiler_params=pltpu.CompilerParams(
            vmem_limit_bytes=100 * 1024 * 1024),
    )(rowm)


def _make_gather():
    mesh = plsc.VectorSubcoreMesh(core_axis_name="c", subcore_axis_name="s")

    @functools.partial(
        pl.kernel, mesh=mesh,
        out_type=jax.ShapeDtypeStruct((4 * GK,), jnp.float32),
        compiler_params=pltpu.CompilerParams(needs_layout_passes=False),
        scratch_types=[
            pltpu.VMEM((GK,), jnp.int32),
            pltpu.VMEM((4 * NP,), jnp.float32),
            pltpu.VMEM((4 * GK,), jnp.float32),
        ],
    )
    def gather_k(idx_hbm, flat_hbm, out_hbm, idx_v, flat_v, out_v):
        cid = lax.axis_index("c")
        sid = lax.axis_index("s")

        @pl.when((cid == 0) & (sid == 0))
        def _():
            pltpu.sync_copy(idx_hbm, idx_v)
            pltpu.sync_copy(flat_hbm.at[pl.ds(0, 4 * NP)], flat_v)
            for i in range(GK // 16):
                iv = idx_v[pl.ds(i * 16, 16)]
                for c in range(4):
                    vals = plsc.load_gather(flat_v, [iv + c * NP])
                    out_v[pl.ds(c * GK + i * 16, 16)] = vals
            pltpu.sync_copy(out_v, out_hbm)

    return gather_k


_permute_fn = None
_gather_fn = None


def _permute_vals(rank, vals):
    global _permute_fn
    if _permute_fn is None:
        _permute_fn = _make_permute()
    return _permute_fn(rank, vals)


def _gather_boxes(idx, flat):
    global _gather_fn
    if _gather_fn is None:
        _gather_fn = _make_gather()
    return _gather_fn(idx, flat)


def kernel(boxes, scores):
    boxes = boxes.astype(jnp.float32)
    scores = scores.astype(jnp.float32)
    padn = NP - N

    zp = jnp.zeros((padn,), jnp.float32)
    vals = jnp.concatenate([
        boxes[:, 0], zp,
        boxes[:, 1], zp,
        boxes[:, 2], zp,
        boxes[:, 3], zp,
        scores, jnp.full((padn,), -1.0, jnp.float32),
    ])
    s_np = vals[4 * NP:]

    rank_i = _rank_call(s_np.reshape(NBLK, NC))

    svals = _permute_vals(rank_i, vals)

    out = _tri_call(svals.reshape(5 * NBLK, NC))

    top_s = out[4, :K]
    idx = out[5, :].astype(jnp.int32)

    rows = _gather_boxes(idx, svals).reshape(4, GK).T
    return jnp.concatenate([rows[:K], top_s[:, None]], axis=1)
